# Initial kernel scaffold; baseline (speedup 1.0000x reference)
import jax
import jax.numpy as jnp
from jax.experimental import pallas as pl


def kernel(h, r, t, mode, rel_emb):
    raise NotImplementedError("write your pallas kernel here")


# speedup vs baseline: 1.3889x; 1.3889x over previous
import functools

import jax
import jax.numpy as jnp
from jax import lax
from jax.experimental import pallas as pl
from jax.experimental.pallas import tpu as pltpu
from jax.experimental.pallas import tpu_sc as plsc

B = 16384
D = 128
L = 16
NC = 2
NS = 16
NW = NC * NS
BPW = B // NW
CH = 128
NCHUNK = BPW // CH

_mesh = plsc.VectorSubcoreMesh(core_axis_name="c", subcore_axis_name="s")


@functools.partial(
    pl.kernel,
    out_type=jax.ShapeDtypeStruct((B,), jnp.float32),
    mesh=_mesh,
    compiler_params=pltpu.CompilerParams(needs_layout_passes=False),
    scratch_types=[
        pltpu.VMEM((CH,), jnp.int32),
        pltpu.VMEM((CH, D), jnp.float32),
        pltpu.VMEM((CH, D), jnp.float32),
        pltpu.VMEM((CH, D), jnp.float32),
        pltpu.VMEM((CH,), jnp.float32),
        pltpu.SemaphoreType.DMA,
    ],
)
def _distmult_sc(h_hbm, r_hbm, t_hbm, rel_hbm, out_hbm,
                 idx_v, h_v, t_v, rel_v, o_v, sem):
    wid = lax.axis_index("s") * NC + lax.axis_index("c")
    base = wid * BPW

    lane = lax.iota(jnp.int32, L)
    last_lane = lane == (L - 1)

    def chunk_body(c, _):
        cbase = base + c * CH
        pltpu.sync_copy(r_hbm.at[pl.ds(cbase, CH)], idx_v)
        cg = pltpu.async_copy(rel_hbm.at[idx_v], rel_v, sem)
        chh = pltpu.async_copy(h_hbm.at[pl.ds(cbase, CH), :], h_v, sem)
        ct = pltpu.async_copy(t_hbm.at[pl.ds(cbase, CH), :], t_v, sem)
        cg.wait()
        chh.wait()
        ct.wait()

        def row(i, _):
            acc = h_v[i, pl.ds(0, L)] * rel_v[i, pl.ds(0, L)] * t_v[i, pl.ds(0, L)]
            for j in range(1, D // L):
                sl = pl.ds(j * L, L)
                acc = acc + h_v[i, sl] * rel_v[i, sl] * t_v[i, sl]
            cs = plsc.cumsum(acc)
            plsc.store_scatter(o_v, [jnp.full((L,), i, jnp.int32)], cs,
                               mask=last_lane)
            return 0

        lax.fori_loop(0, CH, row, 0, unroll=2)
        pltpu.sync_copy(o_v, out_hbm.at[pl.ds(cbase, CH)])
        return 0

    lax.fori_loop(0, NCHUNK, chunk_body, 0)


def kernel(h, r, t, mode, rel_emb):
    del mode
    return _distmult_sc(h, r.astype(jnp.int32), t, rel_emb)


# speedup vs baseline: 1.6187x; 1.1654x over previous
import functools

import jax
import jax.numpy as jnp
from jax import lax
from jax.experimental import pallas as pl
from jax.experimental.pallas import tpu as pltpu
from jax.experimental.pallas import tpu_sc as plsc

B = 16384
D = 128
L = 16
NC = 2
NS = 16
NW = NC * NS
BPW = B // NW
CH = 128
NCHUNK = BPW // CH

_mesh = plsc.VectorSubcoreMesh(core_axis_name="c", subcore_axis_name="s")


@functools.partial(
    pl.kernel,
    out_type=jax.ShapeDtypeStruct((B,), jnp.float32),
    mesh=_mesh,
    compiler_params=pltpu.CompilerParams(needs_layout_passes=False),
    scratch_types=[
        pltpu.VMEM((BPW,), jnp.int32),
        pltpu.VMEM((BPW,), jnp.float32),
        pltpu.VMEM((2, CH, D), jnp.float32),
        pltpu.VMEM((2, CH, D), jnp.float32),
        pltpu.VMEM((2, CH, D), jnp.float32),
        pltpu.SemaphoreType.DMA,
        pltpu.SemaphoreType.DMA,
    ],
)
def _distmult_sc(h_hbm, r_hbm, t_hbm, rel_hbm, out_hbm,
                 idx_v, o_v, h_b, t_b, rel_b, sem0, sem1):
    wid = lax.axis_index("s") * NC + lax.axis_index("c")
    base = wid * BPW
    pltpu.sync_copy(r_hbm.at[pl.ds(base, BPW)], idx_v)

    lane = lax.iota(jnp.int32, L)
    last_lane = lane == (L - 1)
    sems = (sem0, sem1)

    def start(c):
        k = c & 1
        cbase = base + c * CH
        return (
            pltpu.async_copy(rel_hbm.at[idx_v.at[pl.ds(c * CH, CH)]],
                             rel_b.at[k], sems[k]),
            pltpu.async_copy(h_hbm.at[pl.ds(cbase, CH), :], h_b.at[k], sems[k]),
            pltpu.async_copy(t_hbm.at[pl.ds(cbase, CH), :], t_b.at[k], sems[k]),
        )

    pend = start(0)
    for c in range(NCHUNK):
        nxt = start(c + 1) if c + 1 < NCHUNK else None
        for dsc in pend:
            dsc.wait()
        k = c & 1
        hk, tk, rk = h_b.at[k], t_b.at[k], rel_b.at[k]
        obase = c * CH

        def row(i, _):
            acc = hk[i, pl.ds(0, L)] * rk[i, pl.ds(0, L)] * tk[i, pl.ds(0, L)]
            for j in range(1, D // L):
                sl = pl.ds(j * L, L)
                acc = acc + hk[i, sl] * rk[i, sl] * tk[i, sl]
            cs = plsc.cumsum(acc)
            plsc.store_scatter(o_v, [jnp.full((L,), obase + i, jnp.int32)],
                               cs, mask=last_lane)
            return 0

        lax.fori_loop(0, CH, row, 0, unroll=2)
        pend = nxt

    pltpu.sync_copy(o_v, out_hbm.at[pl.ds(base, BPW)])


def kernel(h, r, t, mode, rel_emb):
    del mode
    return _distmult_sc(h, r.astype(jnp.int32), t, rel_emb)
